# Initial kernel scaffold; baseline (speedup 1.0000x reference)
import jax
import jax.numpy as jnp
from jax.experimental import pallas as pl


def kernel(atoms, words, w_pos, w_batch, edge_index, params):
    raise NotImplementedError("write your pallas kernel here")


# speedup vs baseline: 15.0701x; 15.0701x over previous
import functools
import math

import jax
import jax.numpy as jnp
from jax import lax
from jax.experimental import pallas as pl
from jax.experimental.pallas import tpu as pltpu

_N_NODES = 10000
_N_EDGES = 160000
_N_WORDS = 5000
_N_SENTS = 512
_D = 300
_H = 150
_LCAP = 32


def _sig(x):
    return 1.0 / (1.0 + jnp.exp(-x))


def _erf(x):
    a1, a2, a3, a4, a5 = (0.254829592, -0.284496736, 1.421413741,
                          -1.453152027, 1.061405429)
    p = 0.3275911
    ax = jnp.abs(x)
    t = 1.0 / (1.0 + p * ax)
    poly = ((((a5 * t + a4) * t + a3) * t + a2) * t + a1) * t
    e = 1.0 - poly * jnp.exp(-ax * ax)
    return jnp.sign(x) * e


def _gelu(x):
    return 0.5 * x * (1.0 + _erf(x * (1.0 / math.sqrt(2.0))))


def _gru_dir_body(reverse, L_ref, d_ref, w_ref, u_ref, b1_ref, b2_ref, o_ref):
    Lv = L_ref[0]

    def step(s, h):
        t = Lv - 1 - s if reverse else s
        x = d_ref[pl.ds(t, 1)][0]
        gi = jnp.dot(x, w_ref[...], preferred_element_type=jnp.float32) + b1_ref[...]
        gh = jnp.dot(h, u_ref[...], preferred_element_type=jnp.float32) + b2_ref[...]
        r = _sig(gi[:, :_H] + gh[:, :_H])
        z = _sig(gi[:, _H:2 * _H] + gh[:, _H:2 * _H])
        n = jnp.tanh(gi[:, 2 * _H:] + r * gh[:, 2 * _H:])
        h = (1.0 - z) * n + z * h
        o_ref[pl.ds(t, 1)] = h[None]
        return h

    lax.fori_loop(0, Lv, step, jnp.zeros((_N_SENTS, _H), jnp.float32))


def _gru_dir_pallas(dense, L, W, U, bi, bh, reverse):
    vspec = pl.BlockSpec(memory_space=pltpu.VMEM)
    return pl.pallas_call(
        functools.partial(_gru_dir_body, reverse),
        out_shape=jax.ShapeDtypeStruct((_LCAP, _N_SENTS, _H), jnp.float32),
        in_specs=[pl.BlockSpec(memory_space=pltpu.SMEM)] + [vspec] * 5,
        out_specs=vspec,
    )(jnp.asarray(L, jnp.int32).reshape(1), dense,
      W.T, U.T, bi[None], bh[None])


def _fast_word_ctx(word_vecs, flat, L, g):
    dense = (jnp.zeros((_LCAP * _N_SENTS, _D), jnp.float32)
             .at[flat].set(word_vecs)
             .reshape(_LCAP, _N_SENTS, _D))
    fwd = _gru_dir_pallas(dense, L, g['Wih_f'], g['Whh_f'],
                          g['bih_f'], g['bhh_f'], False)
    bwd = _gru_dir_pallas(dense, L, g['Wih_b'], g['Whh_b'],
                          g['bih_b'], g['bhh_b'], True)
    fwd = fwd.reshape(_LCAP * _N_SENTS, _H)[flat]
    bwd = bwd.reshape(_LCAP * _N_SENTS, _H)[flat]
    return jnp.concatenate([fwd, bwd], axis=-1)


def _proj_kernel_body(v_ref, w1_ref, w0_ref, b1_ref, b0_ref, p1_ref, p0_ref):
    v = v_ref[...]
    p1_ref[...] = jnp.dot(v, w1_ref[...], preferred_element_type=jnp.float32) + b1_ref[...]
    p0_ref[...] = jnp.dot(v, w0_ref[...], preferred_element_type=jnp.float32) + b0_ref[...]


_NB = 2000


def _proj_pallas(vecs, W1, W0, b1, b0):
    full = lambda s: pl.BlockSpec(s, lambda i: (0, 0))
    blk = lambda c: pl.BlockSpec((_NB, c), lambda i: (i, 0))
    return pl.pallas_call(
        _proj_kernel_body,
        grid=(_N_NODES // _NB,),
        out_shape=[jax.ShapeDtypeStruct((_N_NODES, 2 * _H), jnp.float32),
                   jax.ShapeDtypeStruct((_N_NODES, 2 * _H), jnp.float32)],
        in_specs=[blk(_D), full((_D, 2 * _H)), full((_D, 2 * _H)),
                  full((1, 2 * _H)), full((1, 2 * _H))],
        out_specs=[blk(2 * _H), blk(2 * _H)],
    )(vecs, W1, W0, b1, b0)


def _agg_kernel_body(v_ref, r_ref, c_ref, wa1_ref, wa2_ref, ba_ref, o_ref):
    u = (jnp.dot(r_ref[...], wa1_ref[...], preferred_element_type=jnp.float32)
         + jnp.dot(c_ref[...], wa2_ref[...], preferred_element_type=jnp.float32)
         + ba_ref[...])
    o_ref[...] = v_ref[...] + _gelu(u)


def _agg_pallas(vecs, r2c, c2r, Wa1, Wa2, ba):
    full = lambda s: pl.BlockSpec(s, lambda i: (0, 0))
    blk = lambda c: pl.BlockSpec((_NB, c), lambda i: (i, 0))
    return pl.pallas_call(
        _agg_kernel_body,
        grid=(_N_NODES // _NB,),
        out_shape=jax.ShapeDtypeStruct((_N_NODES, _D), jnp.float32),
        in_specs=[blk(_D), blk(_H), blk(_H), full((_H, _D)), full((_H, _D)),
                  full((1, _D))],
        out_specs=blk(_D),
    )(vecs, r2c, c2r, Wa1, Wa2, ba)


def kernel(atoms, words, w_pos, w_batch, edge_index, params):
    counts = jnp.bincount(w_batch, length=_N_SENTS)
    offsets = jnp.concatenate(
        [jnp.zeros((1,), counts.dtype), jnp.cumsum(counts)[:-1]])
    pos = (jnp.arange(_N_WORDS, dtype=jnp.int32)
           - offsets[w_batch].astype(jnp.int32))
    L = counts.max().astype(jnp.int32)
    flat = pos * _N_SENTS + w_batch.astype(jnp.int32)

    word_vecs = params['word_table'][words]
    g = params['gru']

    def slow_branch():
        def cell(x, h, W, U, bi, bh):
            gi = x @ W.T + bi
            gh = h @ U.T + bh
            r = _sig(gi[:, :_H] + gh[:, :_H])
            z = _sig(gi[:, _H:2 * _H] + gh[:, _H:2 * _H])
            n = jnp.tanh(gi[:, 2 * _H:] + r * gh[:, 2 * _H:])
            return (1.0 - z) * n + z * h

        def gather_x(t):
            idx = jnp.clip(offsets + t, 0, _N_WORDS - 1)
            x = word_vecs[idx]
            return jnp.where((t < counts)[:, None], x, 0.0)

        def fstep(h, t):
            h = cell(gather_x(t), h, g['Wih_f'], g['Whh_f'], g['bih_f'], g['bhh_f'])
            return h, h

        _, fys = lax.scan(fstep, jnp.zeros((_N_SENTS, _H), jnp.float32),
                          jnp.arange(_N_WORDS, dtype=jnp.int32))

        def bstep(h, s):
            t = L - 1 - s
            tc = jnp.maximum(t, 0)
            x = jnp.where(t >= 0, gather_x(tc), 0.0)
            hn = cell(x, h, g['Wih_b'], g['Whh_b'], g['bih_b'], g['bhh_b'])
            hn = jnp.where(t >= 0, hn, h)
            return hn, hn

        _, bys = lax.scan(bstep, jnp.zeros((_N_SENTS, _H), jnp.float32),
                          jnp.arange(_N_WORDS, dtype=jnp.int32))

        wb = w_batch.astype(jnp.int32)
        fw = fys[pos, wb]
        bw = bys[jnp.clip(L - 1 - pos, 0, _N_WORDS - 1), wb]
        return jnp.concatenate([fw, bw], axis=-1)

    word_ctx = lax.cond(
        L <= _LCAP,
        lambda: _fast_word_ctx(word_vecs, flat, L, g),
        slow_branch,
    )

    vecs = params['atom_table'][atoms].at[w_pos].set(word_ctx)

    e0 = edge_index[0].astype(jnp.int32)
    e1 = edge_index[1].astype(jnp.int32)
    zH = jnp.zeros((_H,), jnp.float32)
    for lyr in params['layers']:
        Wr, Wc = lyr['Wr'], lyr['Wc']
        W1 = jnp.concatenate([Wr[:_D], Wc[_D:]], axis=1)
        W0 = jnp.concatenate([Wr[_D:], Wc[:_D]], axis=1)
        b1 = jnp.concatenate([lyr['br'], zH])[None]
        b0 = jnp.concatenate([zH, lyr['bc']])[None]
        P1, P0 = _proj_pallas(vecs, W1, W0, b1, b0)
        g1 = P1[e1]
        g0 = P0[e0]
        mr = jax.nn.gelu(g1[:, :_H] + g0[:, :_H], approximate=False)
        mc = jax.nn.gelu(g1[:, _H:] + g0[:, _H:], approximate=False)
        r2c = jax.ops.segment_sum(mr, e1, num_segments=_N_NODES)
        c2r = jax.ops.segment_sum(mc, e0, num_segments=_N_NODES)
        vecs = _agg_pallas(vecs, r2c, c2r,
                           params['Wagg'][:_H], params['Wagg'][_H:],
                           params['bagg'][None])
    return vecs


# speedup vs baseline: 23.7094x; 1.5733x over previous
import functools
import math

import jax
import jax.numpy as jnp
from jax import lax
from jax.experimental import pallas as pl
from jax.experimental.pallas import tpu as pltpu
from jax.experimental.pallas import tpu_sc as plsc

_N_NODES = 10000
_N_EDGES = 160000
_N_WORDS = 5000
_N_SENTS = 512
_D = 300
_H = 150
_LCAP = 32


def _sig(x):
    return 1.0 / (1.0 + jnp.exp(-x))


def _erf(x):
    a1, a2, a3, a4, a5 = (0.254829592, -0.284496736, 1.421413741,
                          -1.453152027, 1.061405429)
    p = 0.3275911
    ax = jnp.abs(x)
    t = 1.0 / (1.0 + p * ax)
    poly = ((((a5 * t + a4) * t + a3) * t + a2) * t + a1) * t
    e = 1.0 - poly * jnp.exp(-ax * ax)
    return jnp.sign(x) * e


def _gelu(x):
    return 0.5 * x * (1.0 + _erf(x * (1.0 / math.sqrt(2.0))))


def _gru_dir_body(reverse, L_ref, d_ref, w_ref, u_ref, b1_ref, b2_ref, o_ref):
    Lv = L_ref[0]

    def step(s, h):
        t = Lv - 1 - s if reverse else s
        x = d_ref[pl.ds(t, 1)][0]
        gi = jnp.dot(x, w_ref[...], preferred_element_type=jnp.float32, precision=lax.Precision.HIGHEST) + b1_ref[...]
        gh = jnp.dot(h, u_ref[...], preferred_element_type=jnp.float32, precision=lax.Precision.HIGHEST) + b2_ref[...]
        r = _sig(gi[:, :_H] + gh[:, :_H])
        z = _sig(gi[:, _H:2 * _H] + gh[:, _H:2 * _H])
        n = jnp.tanh(gi[:, 2 * _H:] + r * gh[:, 2 * _H:])
        h = (1.0 - z) * n + z * h
        o_ref[pl.ds(t, 1)] = h[None]
        return h

    lax.fori_loop(0, Lv, step, jnp.zeros((_N_SENTS, _H), jnp.float32))


def _gru_dir_pallas(dense, L, W, U, bi, bh, reverse):
    vspec = pl.BlockSpec(memory_space=pltpu.VMEM)
    return pl.pallas_call(
        functools.partial(_gru_dir_body, reverse),
        out_shape=jax.ShapeDtypeStruct((_LCAP, _N_SENTS, _H), jnp.float32),
        in_specs=[pl.BlockSpec(memory_space=pltpu.SMEM)] + [vspec] * 5,
        out_specs=vspec,
    )(jnp.asarray(L, jnp.int32).reshape(1), dense,
      W.T, U.T, bi[None], bh[None])


def _fast_word_ctx(word_vecs, flat, L, g):
    dense = (jnp.zeros((_LCAP * _N_SENTS, _D), jnp.float32)
             .at[flat].set(word_vecs)
             .reshape(_LCAP, _N_SENTS, _D))
    fwd = _gru_dir_pallas(dense, L, g['Wih_f'], g['Whh_f'],
                          g['bih_f'], g['bhh_f'], False)
    bwd = _gru_dir_pallas(dense, L, g['Wih_b'], g['Whh_b'],
                          g['bih_b'], g['bhh_b'], True)
    fwd = fwd.reshape(_LCAP * _N_SENTS, _H)[flat]
    bwd = bwd.reshape(_LCAP * _N_SENTS, _H)[flat]
    return jnp.concatenate([fwd, bwd], axis=-1)


_NROW = 10240
_HALF = 75
_WPAD = 80
_EPAD = 163840
_NPB = 2048


def _proj_kernel_body(v_ref, wa_ref, wb_ref, ba_ref, bb_ref, a_ref, b_ref):
    v = v_ref[...]
    a_ref[...] = (jnp.dot(v, wa_ref[0], preferred_element_type=jnp.float32, precision=lax.Precision.HIGHEST)
                  + ba_ref[0])[None]
    b_ref[...] = (jnp.dot(v, wb_ref[0], preferred_element_type=jnp.float32, precision=lax.Precision.HIGHEST)
                  + bb_ref[0])[None]


def _proj_pallas(vecs_p, WA, WB, bA, bB):
    vblk = pl.BlockSpec((_NPB, _D), lambda j, i: (i, 0))
    wblk = pl.BlockSpec((1, _D, _WPAD), lambda j, i: (j, 0, 0))
    bblk = pl.BlockSpec((1, 1, _WPAD), lambda j, i: (j, 0, 0))
    oblk = pl.BlockSpec((1, _NPB, _WPAD), lambda j, i: (j, i, 0))
    return pl.pallas_call(
        _proj_kernel_body,
        grid=(4, _NROW // _NPB),
        out_shape=[jax.ShapeDtypeStruct((4, _NROW, _WPAD), jnp.float32),
                   jax.ShapeDtypeStruct((4, _NROW, _WPAD), jnp.float32)],
        in_specs=[vblk, wblk, wblk, bblk, bblk],
        out_specs=[oblk, oblk],
    )(vecs_p, WA, WB, bA, bB)


def _edge_sc(a_all, b_all, e1p, e0p, zrows, half):
    mesh = plsc.VectorSubcoreMesh(core_axis_name="c", subcore_axis_name="s",
                                  num_cores=2)

    @functools.partial(
        pl.kernel, mesh=mesh,
        compiler_params=pltpu.CompilerParams(use_tc_tiling_on_sc=False),
        out_type=jax.ShapeDtypeStruct((2 * _NROW, _WPAD), jnp.float32),
        scratch_types=[
            pltpu.VMEM((128,), jnp.int32),
            pltpu.VMEM((128,), jnp.int32),
            pltpu.VMEM((128,), jnp.int32),
            pltpu.VMEM((128, _WPAD), jnp.float32),
            pltpu.VMEM((128, _WPAD), jnp.float32),
            pltpu.VMEM_SHARED((_NROW, _WPAD), jnp.float32),
            pltpu.SemaphoreType.DMA,
            pltpu.SemaphoreType.DMA,
        ],
    )
    def k(a_hbm, b_hbm, e1_hbm, e0_hbm, z_hbm, out_hbm,
          g1_v, g0_v, sx_v, a_v, b_v, acc, sem_a, sem_b):
        cid = lax.axis_index("c")
        sid = lax.axis_index("s")
        off = (2 * cid + half) * _NROW
        isr = jnp.where(cid == 0, 1, 0)
        stripe = 640 * sid
        pltpu.sync_copy(z_hbm, acc.at[pl.ds(stripe, 640)])
        plsc.subcore_barrier()

        ebase = sid * (_EPAD // 16)

        def chunk(j, carry):
            base = ebase + j * 128
            pltpu.sync_copy(e1_hbm.at[pl.ds(base, 128)], g1_v)
            pltpu.sync_copy(e0_hbm.at[pl.ds(base, 128)], g0_v)

            def fix(kk, c2):
                dsl = pl.ds(kk * 16, 16)
                v1 = g1_v[dsl]
                v0 = g0_v[dsl]
                sx_v[dsl] = v1 * isr + v0 * (1 - isr)
                g1_v[dsl] = v1 + off
                g0_v[dsl] = v0 + off
                return c2

            lax.fori_loop(0, 8, fix, 0)

            cpa = pltpu.async_copy(a_hbm.at[g1_v], a_v, sem_a)
            cpb = pltpu.async_copy(b_hbm.at[g0_v], b_v, sem_b)
            cpa.wait()
            cpb.wait()

            def row(r, c2):
                for kk in range(_WPAD // 16):
                    dsl = pl.ds(kk * 16, 16)
                    a_v[r, dsl] = _gelu(a_v[r, dsl] + b_v[r, dsl])
                return c2

            lax.fori_loop(0, 128, row, 0)

            pltpu.sync_copy(a_v, acc.at[sx_v], add=True)
            return carry

        lax.fori_loop(0, _EPAD // 16 // 128, chunk, 0)
        plsc.subcore_barrier()
        pltpu.sync_copy(acc.at[pl.ds(stripe, 640)],
                        out_hbm.at[pl.ds(cid * _NROW + stripe, 640)])

    return k(a_all, b_all, e1p, e0p, zrows)


def _pad_w(w, h):
    wh = w[:, h * _HALF:(h + 1) * _HALF]
    return jnp.concatenate(
        [wh, jnp.zeros((_D, _WPAD - _HALF), jnp.float32)], 1)


def _pad_b(b, h):
    bh = b[h * _HALF:(h + 1) * _HALF]
    return jnp.concatenate(
        [bh, jnp.zeros((_WPAD - _HALF,), jnp.float32)])[None]


def _edge_stage(vecs, Wr, Wc, br, bc, e1p, e0p, zrows):
    vp = jnp.zeros((_NROW, _D), jnp.float32).at[:_N_NODES].set(vecs)
    zb = jnp.zeros((1, _WPAD), jnp.float32)
    WA = jnp.stack([_pad_w(Wr[:_D], 0), _pad_w(Wr[:_D], 1),
                    _pad_w(Wc[_D:], 0), _pad_w(Wc[_D:], 1)])
    WB = jnp.stack([_pad_w(Wr[_D:], 0), _pad_w(Wr[_D:], 1),
                    _pad_w(Wc[:_D], 0), _pad_w(Wc[:_D], 1)])
    bA = jnp.stack([_pad_b(br, 0), _pad_b(br, 1), zb, zb])
    bB = jnp.stack([zb, zb, _pad_b(bc, 0), _pad_b(bc, 1)])
    A3, B3 = _proj_pallas(vp, WA, WB, bA, bB)
    af = A3.reshape(4 * _NROW, _WPAD)
    bf = B3.reshape(4 * _NROW, _WPAD)
    rr0 = _edge_sc(af, bf, e1p, e0p, zrows, 0)
    rr1 = _edge_sc(af, bf, e1p, e0p, zrows, 1)
    r2c = jnp.concatenate([rr0[:_N_NODES, :_HALF], rr1[:_N_NODES, :_HALF]], 1)
    c2r = jnp.concatenate([rr0[_NROW:_NROW + _N_NODES, :_HALF],
                           rr1[_NROW:_NROW + _N_NODES, :_HALF]], 1)
    return r2c, c2r


_NB = 2000


def _agg_kernel_body(v_ref, r_ref, c_ref, wa1_ref, wa2_ref, ba_ref, o_ref):
    u = (jnp.dot(r_ref[...], wa1_ref[...], preferred_element_type=jnp.float32, precision=lax.Precision.HIGHEST)
         + jnp.dot(c_ref[...], wa2_ref[...], preferred_element_type=jnp.float32, precision=lax.Precision.HIGHEST)
         + ba_ref[...])
    o_ref[...] = v_ref[...] + _gelu(u)


def _agg_pallas(vecs, r2c, c2r, Wa1, Wa2, ba):
    full = lambda s: pl.BlockSpec(s, lambda i: (0, 0))
    blk = lambda c: pl.BlockSpec((_NB, c), lambda i: (i, 0))
    return pl.pallas_call(
        _agg_kernel_body,
        grid=(_N_NODES // _NB,),
        out_shape=jax.ShapeDtypeStruct((_N_NODES, _D), jnp.float32),
        in_specs=[blk(_D), blk(_H), blk(_H), full((_H, _D)), full((_H, _D)),
                  full((1, _D))],
        out_specs=blk(_D),
    )(vecs, r2c, c2r, Wa1, Wa2, ba)


def kernel(atoms, words, w_pos, w_batch, edge_index, params):
    counts = jnp.bincount(w_batch, length=_N_SENTS)
    offsets = jnp.concatenate(
        [jnp.zeros((1,), counts.dtype), jnp.cumsum(counts)[:-1]])
    pos = (jnp.arange(_N_WORDS, dtype=jnp.int32)
           - offsets[w_batch].astype(jnp.int32))
    L = counts.max().astype(jnp.int32)
    flat = pos * _N_SENTS + w_batch.astype(jnp.int32)

    word_vecs = params['word_table'][words]
    g = params['gru']

    def slow_branch():
        def cell(x, h, W, U, bi, bh):
            gi = x @ W.T + bi
            gh = h @ U.T + bh
            r = _sig(gi[:, :_H] + gh[:, :_H])
            z = _sig(gi[:, _H:2 * _H] + gh[:, _H:2 * _H])
            n = jnp.tanh(gi[:, 2 * _H:] + r * gh[:, 2 * _H:])
            return (1.0 - z) * n + z * h

        def gather_x(t):
            idx = jnp.clip(offsets + t, 0, _N_WORDS - 1)
            x = word_vecs[idx]
            return jnp.where((t < counts)[:, None], x, 0.0)

        def fstep(h, t):
            h = cell(gather_x(t), h, g['Wih_f'], g['Whh_f'], g['bih_f'], g['bhh_f'])
            return h, h

        _, fys = lax.scan(fstep, jnp.zeros((_N_SENTS, _H), jnp.float32),
                          jnp.arange(_N_WORDS, dtype=jnp.int32))

        def bstep(h, s):
            t = L - 1 - s
            tc = jnp.maximum(t, 0)
            x = jnp.where(t >= 0, gather_x(tc), 0.0)
            hn = cell(x, h, g['Wih_b'], g['Whh_b'], g['bih_b'], g['bhh_b'])
            hn = jnp.where(t >= 0, hn, h)
            return hn, hn

        _, bys = lax.scan(bstep, jnp.zeros((_N_SENTS, _H), jnp.float32),
                          jnp.arange(_N_WORDS, dtype=jnp.int32))

        wb = w_batch.astype(jnp.int32)
        fw = fys[pos, wb]
        bw = bys[jnp.clip(L - 1 - pos, 0, _N_WORDS - 1), wb]
        return jnp.concatenate([fw, bw], axis=-1)

    word_ctx = lax.cond(
        L <= _LCAP,
        lambda: _fast_word_ctx(word_vecs, flat, L, g),
        slow_branch,
    )

    vecs = params['atom_table'][atoms].at[w_pos].set(word_ctx)

    e0 = edge_index[0].astype(jnp.int32)
    e1 = edge_index[1].astype(jnp.int32)
    npad = _EPAD - _N_EDGES
    e1p = jnp.concatenate([e1, jnp.full((npad,), _N_NODES, jnp.int32)])
    e0p = jnp.concatenate([e0, jnp.full((npad,), _N_NODES, jnp.int32)])
    zrows = jnp.zeros((640, _WPAD), jnp.float32)
    for lyr in params['layers']:
        r2c, c2r = _edge_stage(vecs, lyr['Wr'], lyr['Wc'], lyr['br'],
                               lyr['bc'], e1p, e0p, zrows)
        vecs = _agg_pallas(vecs, r2c, c2r,
                           params['Wagg'][:_H], params['Wagg'][_H:],
                           params['bagg'][None])
    return vecs


# speedup vs baseline: 28.0040x; 1.1811x over previous
import functools
import math

import jax
import jax.numpy as jnp
from jax import lax
from jax.experimental import pallas as pl
from jax.experimental.pallas import tpu as pltpu
from jax.experimental.pallas import tpu_sc as plsc

_N_NODES = 10000
_N_EDGES = 160000
_N_WORDS = 5000
_N_SENTS = 512
_D = 300
_H = 150
_LCAP = 32


def _sig(x):
    return 1.0 / (1.0 + jnp.exp(-x))


def _erf(x):
    a1, a2, a3, a4, a5 = (0.254829592, -0.284496736, 1.421413741,
                          -1.453152027, 1.061405429)
    p = 0.3275911
    ax = jnp.abs(x)
    t = 1.0 / (1.0 + p * ax)
    poly = ((((a5 * t + a4) * t + a3) * t + a2) * t + a1) * t
    e = 1.0 - poly * jnp.exp(-ax * ax)
    return jnp.sign(x) * e


def _gelu(x):
    return 0.5 * x * (1.0 + _erf(x * (1.0 / math.sqrt(2.0))))


def _gru_dir_body(reverse, L_ref, d_ref, w_ref, u_ref, b1_ref, b2_ref, o_ref):
    Lv = L_ref[0]

    def step(s, h):
        t = Lv - 1 - s if reverse else s
        x = d_ref[pl.ds(t, 1)][0]
        gi = jnp.dot(x, w_ref[...], preferred_element_type=jnp.float32, precision=lax.Precision.HIGHEST) + b1_ref[...]
        gh = jnp.dot(h, u_ref[...], preferred_element_type=jnp.float32, precision=lax.Precision.HIGHEST) + b2_ref[...]
        r = _sig(gi[:, :_H] + gh[:, :_H])
        z = _sig(gi[:, _H:2 * _H] + gh[:, _H:2 * _H])
        n = jnp.tanh(gi[:, 2 * _H:] + r * gh[:, 2 * _H:])
        h = (1.0 - z) * n + z * h
        o_ref[pl.ds(t, 1)] = h[None]
        return h

    lax.fori_loop(0, Lv, step, jnp.zeros((_N_SENTS, _H), jnp.float32))


def _gru_dir_pallas(dense, L, W, U, bi, bh, reverse):
    vspec = pl.BlockSpec(memory_space=pltpu.VMEM)
    return pl.pallas_call(
        functools.partial(_gru_dir_body, reverse),
        out_shape=jax.ShapeDtypeStruct((_LCAP, _N_SENTS, _H), jnp.float32),
        in_specs=[pl.BlockSpec(memory_space=pltpu.SMEM)] + [vspec] * 5,
        out_specs=vspec,
    )(jnp.asarray(L, jnp.int32).reshape(1), dense,
      W.T, U.T, bi[None], bh[None])


def _fast_word_ctx(word_vecs, flat, L, g):
    dense = (jnp.zeros((_LCAP * _N_SENTS, _D), jnp.float32)
             .at[flat].set(word_vecs)
             .reshape(_LCAP, _N_SENTS, _D))
    fwd = _gru_dir_pallas(dense, L, g['Wih_f'], g['Whh_f'],
                          g['bih_f'], g['bhh_f'], False)
    bwd = _gru_dir_pallas(dense, L, g['Wih_b'], g['Whh_b'],
                          g['bih_b'], g['bhh_b'], True)
    fwd = fwd.reshape(_LCAP * _N_SENTS, _H)[flat]
    bwd = bwd.reshape(_LCAP * _N_SENTS, _H)[flat]
    return jnp.concatenate([fwd, bwd], axis=-1)


_NROW = 10240
_HALF = 75
_WPAD = 80
_EPAD = 163840
_NPB = 2048


def _proj_kernel_body(v_ref, wa_ref, wb_ref, ba_ref, bb_ref, a_ref, b_ref):
    v = v_ref[...]
    a_ref[...] = (jnp.dot(v, wa_ref[0], preferred_element_type=jnp.float32, precision=lax.Precision.HIGHEST)
                  + ba_ref[0])[None]
    b_ref[...] = (jnp.dot(v, wb_ref[0], preferred_element_type=jnp.float32, precision=lax.Precision.HIGHEST)
                  + bb_ref[0])[None]


def _proj_pallas(vecs_p, WA, WB, bA, bB):
    vblk = pl.BlockSpec((_NPB, _D), lambda j, i: (i, 0))
    wblk = pl.BlockSpec((1, _D, _WPAD), lambda j, i: (j, 0, 0))
    bblk = pl.BlockSpec((1, 1, _WPAD), lambda j, i: (j, 0, 0))
    oblk = pl.BlockSpec((1, _NPB, _WPAD), lambda j, i: (j, i, 0))
    return pl.pallas_call(
        _proj_kernel_body,
        grid=(4, _NROW // _NPB),
        out_shape=[jax.ShapeDtypeStruct((4, _NROW, _WPAD), jnp.float32),
                   jax.ShapeDtypeStruct((4, _NROW, _WPAD), jnp.float32)],
        in_specs=[vblk, wblk, wblk, bblk, bblk],
        out_specs=[oblk, oblk],
    )(vecs_p, WA, WB, bA, bB)


def _edge_sc(a_all, b_all, epairs, zrows, half):
    mesh = plsc.VectorSubcoreMesh(core_axis_name="c", subcore_axis_name="s",
                                  num_cores=2)

    @functools.partial(
        pl.kernel, mesh=mesh,
        compiler_params=pltpu.CompilerParams(use_tc_tiling_on_sc=False),
        out_type=jax.ShapeDtypeStruct((2 * _NROW, _WPAD), jnp.float32),
        scratch_types=[
            pltpu.VMEM((1, 256), jnp.int32),
            pltpu.VMEM((1, 256), jnp.int32),
            pltpu.VMEM((128,), jnp.int32),
            pltpu.VMEM((128,), jnp.int32),
            pltpu.VMEM((128,), jnp.int32),
            pltpu.VMEM((128,), jnp.int32),
            pltpu.VMEM((128,), jnp.int32),
            pltpu.VMEM((128,), jnp.int32),
            pltpu.VMEM((128, _WPAD), jnp.float32),
            pltpu.VMEM((128, _WPAD), jnp.float32),
            pltpu.VMEM((128, _WPAD), jnp.float32),
            pltpu.VMEM((128, _WPAD), jnp.float32),
            pltpu.VMEM_SHARED((_NROW, _WPAD), jnp.float32),
            pltpu.SemaphoreType.DMA,
            pltpu.SemaphoreType.DMA,
            pltpu.SemaphoreType.DMA,
            pltpu.SemaphoreType.DMA,
        ],
    )
    def k(a_hbm, b_hbm, ep_hbm, z_hbm, out_hbm,
          g01a, g01b, g1a, g0a, sxa, g1b, g0b, sxb,
          ava, bva, avb, bvb, acc, sem0, sem1, sem2, sem3):
        cid = lax.axis_index("c")
        sid = lax.axis_index("s")
        off = (2 * cid + half) * _NROW
        isr = jnp.where(cid == 0, 1, 0)
        stripe = 640 * sid
        pltpu.sync_copy(z_hbm, acc.at[pl.ds(stripe, 640)])
        plsc.subcore_barrier()

        cbase = sid * (_EPAD // 16 // 128)

        def start(row, g01, g1v, g0v, sxv, av, bv, sa, sb):
            pltpu.sync_copy(ep_hbm.at[pl.ds(row, 1)], g01)

            def fix(kk, c2):
                dsl = pl.ds(kk * 16, 16)
                v1 = g01[0, dsl]
                v0 = g01[0, pl.ds(128 + kk * 16, 16)]
                sxv[dsl] = v1 * isr + v0 * (1 - isr)
                g1v[dsl] = v1 + off
                g0v[dsl] = v0 + off
                return c2

            lax.fori_loop(0, 8, fix, 0)
            return (pltpu.async_copy(a_hbm.at[g1v], av, sa),
                    pltpu.async_copy(b_hbm.at[g0v], bv, sb))

        def gelu_rows(av, bv):
            def row(r, c2):
                for kk in range(_WPAD // 16):
                    dsl = pl.ds(kk * 16, 16)
                    av[r, dsl] = _gelu(av[r, dsl] + bv[r, dsl])
                return c2

            lax.fori_loop(0, 128, row, 0)

        def pair(jj, carry):
            row_a = cbase + 2 * jj
            cpa, cpb = start(row_a, g01a, g1a, g0a, sxa, ava, bva, sem0, sem1)
            cqa, cqb = start(row_a + 1, g01b, g1b, g0b, sxb, avb, bvb,
                             sem2, sem3)
            cpa.wait()
            cpb.wait()
            gelu_rows(ava, bva)
            pltpu.sync_copy(ava, acc.at[sxa], add=True)
            cqa.wait()
            cqb.wait()
            gelu_rows(avb, bvb)
            pltpu.sync_copy(avb, acc.at[sxb], add=True)
            return carry

        lax.fori_loop(0, _EPAD // 16 // 256, pair, 0)
        plsc.subcore_barrier()
        pltpu.sync_copy(acc.at[pl.ds(stripe, 640)],
                        out_hbm.at[pl.ds(cid * _NROW + stripe, 640)])

    return k(a_all, b_all, epairs, zrows)


def _pad_w(w, h):
    wh = w[:, h * _HALF:(h + 1) * _HALF]
    return jnp.concatenate(
        [wh, jnp.zeros((_D, _WPAD - _HALF), jnp.float32)], 1)


def _pad_b(b, h):
    bh = b[h * _HALF:(h + 1) * _HALF]
    return jnp.concatenate(
        [bh, jnp.zeros((_WPAD - _HALF,), jnp.float32)])[None]


def _edge_stage(vecs, Wr, Wc, br, bc, epairs, zrows):
    vp = jnp.zeros((_NROW, _D), jnp.float32).at[:_N_NODES].set(vecs)
    zb = jnp.zeros((1, _WPAD), jnp.float32)
    WA = jnp.stack([_pad_w(Wr[:_D], 0), _pad_w(Wr[:_D], 1),
                    _pad_w(Wc[_D:], 0), _pad_w(Wc[_D:], 1)])
    WB = jnp.stack([_pad_w(Wr[_D:], 0), _pad_w(Wr[_D:], 1),
                    _pad_w(Wc[:_D], 0), _pad_w(Wc[:_D], 1)])
    bA = jnp.stack([_pad_b(br, 0), _pad_b(br, 1), zb, zb])
    bB = jnp.stack([zb, zb, _pad_b(bc, 0), _pad_b(bc, 1)])
    A3, B3 = _proj_pallas(vp, WA, WB, bA, bB)
    af = A3.reshape(4 * _NROW, _WPAD)
    bf = B3.reshape(4 * _NROW, _WPAD)
    rr0 = _edge_sc(af, bf, epairs, zrows, 0)
    rr1 = _edge_sc(af, bf, epairs, zrows, 1)
    r2c = jnp.concatenate([rr0[:_N_NODES, :_HALF], rr1[:_N_NODES, :_HALF]], 1)
    c2r = jnp.concatenate([rr0[_NROW:_NROW + _N_NODES, :_HALF],
                           rr1[_NROW:_NROW + _N_NODES, :_HALF]], 1)
    return r2c, c2r


_NB = 2000


def _agg_kernel_body(v_ref, r_ref, c_ref, wa1_ref, wa2_ref, ba_ref, o_ref):
    u = (jnp.dot(r_ref[...], wa1_ref[...], preferred_element_type=jnp.float32, precision=lax.Precision.HIGHEST)
         + jnp.dot(c_ref[...], wa2_ref[...], preferred_element_type=jnp.float32, precision=lax.Precision.HIGHEST)
         + ba_ref[...])
    o_ref[...] = v_ref[...] + _gelu(u)


def _agg_pallas(vecs, r2c, c2r, Wa1, Wa2, ba):
    full = lambda s: pl.BlockSpec(s, lambda i: (0, 0))
    blk = lambda c: pl.BlockSpec((_NB, c), lambda i: (i, 0))
    return pl.pallas_call(
        _agg_kernel_body,
        grid=(_N_NODES // _NB,),
        out_shape=jax.ShapeDtypeStruct((_N_NODES, _D), jnp.float32),
        in_specs=[blk(_D), blk(_H), blk(_H), full((_H, _D)), full((_H, _D)),
                  full((1, _D))],
        out_specs=blk(_D),
    )(vecs, r2c, c2r, Wa1, Wa2, ba)


def kernel(atoms, words, w_pos, w_batch, edge_index, params):
    counts = jnp.bincount(w_batch, length=_N_SENTS)
    offsets = jnp.concatenate(
        [jnp.zeros((1,), counts.dtype), jnp.cumsum(counts)[:-1]])
    pos = (jnp.arange(_N_WORDS, dtype=jnp.int32)
           - offsets[w_batch].astype(jnp.int32))
    L = counts.max().astype(jnp.int32)
    flat = pos * _N_SENTS + w_batch.astype(jnp.int32)

    word_vecs = params['word_table'][words]
    g = params['gru']

    def slow_branch():
        def cell(x, h, W, U, bi, bh):
            gi = x @ W.T + bi
            gh = h @ U.T + bh
            r = _sig(gi[:, :_H] + gh[:, :_H])
            z = _sig(gi[:, _H:2 * _H] + gh[:, _H:2 * _H])
            n = jnp.tanh(gi[:, 2 * _H:] + r * gh[:, 2 * _H:])
            return (1.0 - z) * n + z * h

        def gather_x(t):
            idx = jnp.clip(offsets + t, 0, _N_WORDS - 1)
            x = word_vecs[idx]
            return jnp.where((t < counts)[:, None], x, 0.0)

        def fstep(h, t):
            h = cell(gather_x(t), h, g['Wih_f'], g['Whh_f'], g['bih_f'], g['bhh_f'])
            return h, h

        _, fys = lax.scan(fstep, jnp.zeros((_N_SENTS, _H), jnp.float32),
                          jnp.arange(_N_WORDS, dtype=jnp.int32))

        def bstep(h, s):
            t = L - 1 - s
            tc = jnp.maximum(t, 0)
            x = jnp.where(t >= 0, gather_x(tc), 0.0)
            hn = cell(x, h, g['Wih_b'], g['Whh_b'], g['bih_b'], g['bhh_b'])
            hn = jnp.where(t >= 0, hn, h)
            return hn, hn

        _, bys = lax.scan(bstep, jnp.zeros((_N_SENTS, _H), jnp.float32),
                          jnp.arange(_N_WORDS, dtype=jnp.int32))

        wb = w_batch.astype(jnp.int32)
        fw = fys[pos, wb]
        bw = bys[jnp.clip(L - 1 - pos, 0, _N_WORDS - 1), wb]
        return jnp.concatenate([fw, bw], axis=-1)

    word_ctx = lax.cond(
        L <= _LCAP,
        lambda: _fast_word_ctx(word_vecs, flat, L, g),
        slow_branch,
    )

    vecs = params['atom_table'][atoms].at[w_pos].set(word_ctx)

    e0 = edge_index[0].astype(jnp.int32)
    e1 = edge_index[1].astype(jnp.int32)
    npad = _EPAD - _N_EDGES
    e1p = jnp.concatenate([e1, jnp.full((npad,), _N_NODES, jnp.int32)])
    e0p = jnp.concatenate([e0, jnp.full((npad,), _N_NODES, jnp.int32)])
    epairs = jnp.concatenate(
        [e1p.reshape(-1, 128), e0p.reshape(-1, 128)], axis=1)
    zrows = jnp.zeros((640, _WPAD), jnp.float32)
    for lyr in params['layers']:
        r2c, c2r = _edge_stage(vecs, lyr['Wr'], lyr['Wc'], lyr['br'],
                               lyr['bc'], epairs, zrows)
        vecs = _agg_pallas(vecs, r2c, c2r,
                           params['Wagg'][:_H], params['Wagg'][_H:],
                           params['bagg'][None])
    return vecs
